# baseline (speedup 1.0000x reference)
import functools

import jax
import jax.numpy as jnp
from jax import lax
from jax.experimental import pallas as pl
from jax.experimental.pallas import tpu as pltpu
from jax.experimental.pallas import tpu_sc as plsc

_VOCAB = 8192
_DIM = 256
_BETA = 0.25

_TM = 2048
_TK = 8192


def _argmin_body(x_ref, dict_ref, counts_ref, e2_ref, idx_ref, loss_ref, e_out_ref,
                 bv_ref, bi_ref, iota_ref):
    k = pl.program_id(1)
    nk = pl.num_programs(1)
    i = pl.program_id(0)

    @pl.when((k == 0) & (i == 0))
    def _():
        iota_ref[:, :] = lax.broadcasted_iota(
            jnp.int32, (1, 128), 1).astype(jnp.float32)
        e_out_ref[:, :] = dict_ref[:, :] / counts_ref[:, :]
    nj = _TK // 128
    rb = 512
    lv_parts = []
    li_parts = []
    for r in range(_TM // rb):
        rs = slice(r * rb, (r + 1) * rb)
        sr = lax.dot_general(
            -2.0 * x_ref[rs, :], e_out_ref[:, :], (((1,), (1,)), ((), ())),
            preferred_element_type=jnp.float32,
        )
        xr = x_ref[rs, :]
        x2r = jnp.sum(xr * xr, axis=1, keepdims=True)
        acc_v = (x2r + sr[:, 0:128]) + e2_ref[:, 0:128]
        acc_j = jnp.zeros_like(acc_v)
        for j in range(1, nj):
            dj = (x2r + sr[:, j * 128:(j + 1) * 128]) + e2_ref[:, j * 128:(j + 1) * 128]
            m = dj < acc_v
            acc_v = jnp.where(m, dj, acc_v)
            acc_j = jnp.where(m, jnp.float32(j), acc_j)
        lvr = jnp.min(acc_v, axis=1, keepdims=True)
        idxf = acc_j * 128.0 + iota_ref[:, :]
        lir = jnp.min(jnp.where(acc_v == lvr, idxf, jnp.float32(_VOCAB)),
                      axis=1, keepdims=True)
        lv_parts.append(lvr)
        li_parts.append(lir)
    lv = jnp.concatenate(lv_parts, axis=0)
    li = jnp.concatenate(li_parts, axis=0) + jnp.float32(k * _TK)

    @pl.when(k == 0)
    def _():
        bv_ref[:, :] = lv
        bi_ref[:, :] = li

    @pl.when(k > 0)
    def _():
        take = lv < bv_ref[:, :]
        bi_ref[:, :] = jnp.where(take, li, bi_ref[:, :])
        bv_ref[:, :] = jnp.where(take, lv, bv_ref[:, :])

    @pl.when(k == nk - 1)
    def _():
        idx_ref[:, :] = bi_ref[:, :].astype(jnp.int32)
        part = jnp.sum(bv_ref[:, :], keepdims=True)
        i = pl.program_id(0)

        @pl.when(i == 0)
        def _():
            loss_ref[:, :] = part

        @pl.when(i > 0)
        def _():
            loss_ref[:, :] = loss_ref[:, :] + part


def _argmin_call(x, dictionary, counts_col, e2):
    m = x.shape[0]
    grid = (m // _TM, _VOCAB // _TK)
    return pl.pallas_call(
        _argmin_body,
        grid=grid,
        in_specs=[
            pl.BlockSpec((_TM, _DIM), lambda i, k: (i, 0)),
            pl.BlockSpec((_TK, _DIM), lambda i, k: (k, 0)),
            pl.BlockSpec((_TK, 1), lambda i, k: (k, 0)),
            pl.BlockSpec((1, _TK), lambda i, k: (0, k)),
        ],
        out_specs=[
            pl.BlockSpec((_TM, 1), lambda i, k: (i, 0)),
            pl.BlockSpec((1, 1), lambda i, k: (0, 0)),
            pl.BlockSpec((_TK, _DIM), lambda i, k: (k, 0)),
        ],
        out_shape=[
            jax.ShapeDtypeStruct((m, 1), jnp.int32),
            jax.ShapeDtypeStruct((1, 1), jnp.float32),
            jax.ShapeDtypeStruct((_VOCAB, _DIM), jnp.float32),
        ],
        scratch_shapes=[
            pltpu.VMEM((_TM, 1), jnp.float32),
            pltpu.VMEM((_TM, 1), jnp.float32),
            pltpu.VMEM((1, 128), jnp.float32),
        ],
    )(x, dictionary, counts_col, e2)


def _perp_body(h_ref, out_ref):
    c = h_ref[0:1, :] + h_ref[1:2, :]
    p = c * (1.0 / 8192.0)
    t = p * jnp.log(p + 1e-10)
    out_ref[:, :] = jnp.exp(-jnp.sum(t, axis=1, keepdims=True))


def _perp_call(hist):
    return pl.pallas_call(
        _perp_body,
        out_shape=jax.ShapeDtypeStruct((1, 1), jnp.float32),
    )(hist)


def _sc_gather_hist(e, idx, zeros_k, ones_h):
    info = plsc.get_sparse_core_info()
    nc, ns = info.num_cores, info.num_subcores
    nw = nc * ns
    m = idx.shape[0]
    bpw = m // nw
    half = bpw // 2
    mesh = plsc.VectorSubcoreMesh(core_axis_name="c", subcore_axis_name="s")

    @functools.partial(
        pl.kernel,
        mesh=mesh,
        out_type=[
            jax.ShapeDtypeStruct((m, _DIM), jnp.float32),
            jax.ShapeDtypeStruct((nc, _VOCAB), jnp.float32),
        ],
        scratch_types=[
            pltpu.VMEM((2, half), jnp.int32),
            pltpu.VMEM((half, _DIM), jnp.float32),
            pltpu.VMEM((half, _DIM), jnp.float32),
            pltpu.VMEM((half,), jnp.float32),
            pltpu.VMEM_SHARED((_VOCAB,), jnp.float32),
            pltpu.SemaphoreType.DMA,
            pltpu.SemaphoreType.DMA,
            pltpu.SemaphoreType.DMA,
            pltpu.SemaphoreType.DMA,
        ],
    )
    def body(e_hbm, idx_hbm, zeros_hbm, ones_hbm, q_hbm, hist_hbm,
             idx_v, rows_a, rows_b, ones_v, hist_s, sem_a, sem_b,
             sem_wa, sem_wb):
        cid = lax.axis_index("c")
        sid = lax.axis_index("s")
        wid = sid * nc + cid
        base = wid * bpw
        pltpu.sync_copy(idx_hbm.at[pl.ds(base, half)], idx_v.at[0])
        pltpu.sync_copy(idx_hbm.at[pl.ds(base + half, half)], idx_v.at[1])
        cp_a = pltpu.async_copy(e_hbm.at[idx_v.at[0]], rows_a, sem_a)
        cp_b = pltpu.async_copy(e_hbm.at[idx_v.at[1]], rows_b, sem_b)
        pltpu.sync_copy(ones_hbm, ones_v)

        @pl.when(sid == 0)
        def _():
            pltpu.sync_copy(zeros_hbm, hist_s)

        cp_a.wait()
        wr_a = pltpu.async_copy(rows_a, q_hbm.at[pl.ds(base, half)], sem_wa)
        cp_b.wait()
        wr_b = pltpu.async_copy(rows_b, q_hbm.at[pl.ds(base + half, half)], sem_wb)
        plsc.subcore_barrier()
        pltpu.sync_copy(ones_v, hist_s.at[idx_v.at[0]], add=True)
        pltpu.sync_copy(ones_v, hist_s.at[idx_v.at[1]], add=True)
        plsc.subcore_barrier()

        @pl.when(sid == 0)
        def _():
            pltpu.sync_copy(hist_s, hist_hbm.at[cid])

        wr_a.wait()
        wr_b.wait()

    return body(e, idx, zeros_k, ones_h)


def kernel(inputs, dictionary, counts):
    c = inputs.shape[-1]
    x = inputs.reshape(-1, c)
    m = x.shape[0]
    e2 = ((dictionary / counts[:, None]) ** 2).sum(axis=1).reshape(1, -1)

    idx, loss_sum, e = _argmin_call(x, dictionary, counts[:, None], e2)

    zeros_k = jnp.zeros((_VOCAB,), jnp.float32)
    ones_h = jnp.ones((m // 64,), jnp.float32)
    quantized, hist = _sc_gather_hist(e, idx.reshape(m), zeros_k, ones_h)

    perp = _perp_call(hist)

    loss_vq = loss_sum.reshape(()) * ((1.0 + _BETA) / (m * c))
    return (quantized.reshape(inputs.shape), loss_vq, perp.reshape(()))


# speedup vs baseline: 1.0748x; 1.0748x over previous
import functools

import jax
import jax.numpy as jnp
from jax import lax
from jax.experimental import pallas as pl
from jax.experimental.pallas import tpu as pltpu
from jax.experimental.pallas import tpu_sc as plsc

_VOCAB = 8192
_DIM = 256
_BETA = 0.25

_TM = 2048
_TK = 8192


def _argmin_body(x_ref, e_ref, e2_ref, idx_ref, loss_ref,
                 bv_ref, bi_ref, iota_ref):
    k = pl.program_id(1)
    nk = pl.num_programs(1)
    i = pl.program_id(0)

    @pl.when((k == 0) & (i == 0))
    def _():
        iota_ref[:, :] = lax.broadcasted_iota(
            jnp.int32, (1, 128), 1).astype(jnp.float32)
    nj = _TK // 128
    rb = 512
    lv_parts = []
    li_parts = []
    for r in range(_TM // rb):
        rs = slice(r * rb, (r + 1) * rb)
        sr = lax.dot_general(
            -2.0 * x_ref[rs, :], e_ref[:, :], (((1,), (1,)), ((), ())),
            preferred_element_type=jnp.float32,
        )
        xr = x_ref[rs, :]
        x2r = jnp.sum(xr * xr, axis=1, keepdims=True)
        acc_v = (x2r + sr[:, 0:128]) + e2_ref[:, 0:128]
        acc_j = jnp.zeros_like(acc_v)
        for j in range(1, nj):
            dj = (x2r + sr[:, j * 128:(j + 1) * 128]) + e2_ref[:, j * 128:(j + 1) * 128]
            m = dj < acc_v
            acc_v = jnp.where(m, dj, acc_v)
            acc_j = jnp.where(m, jnp.float32(j), acc_j)
        lvr = jnp.min(acc_v, axis=1, keepdims=True)
        idxf = acc_j * 128.0 + iota_ref[:, :]
        lir = jnp.min(jnp.where(acc_v == lvr, idxf, jnp.float32(_VOCAB)),
                      axis=1, keepdims=True)
        lv_parts.append(lvr)
        li_parts.append(lir)
    lv = jnp.concatenate(lv_parts, axis=0)
    li = jnp.concatenate(li_parts, axis=0) + jnp.float32(k * _TK)

    @pl.when(k == 0)
    def _():
        bv_ref[:, :] = lv
        bi_ref[:, :] = li

    @pl.when(k > 0)
    def _():
        take = lv < bv_ref[:, :]
        bi_ref[:, :] = jnp.where(take, li, bi_ref[:, :])
        bv_ref[:, :] = jnp.where(take, lv, bv_ref[:, :])

    @pl.when(k == nk - 1)
    def _():
        idx_ref[:, :] = bi_ref[:, :].astype(jnp.int32)
        part = jnp.sum(bv_ref[:, :], keepdims=True)
        i = pl.program_id(0)

        @pl.when(i == 0)
        def _():
            loss_ref[:, :] = part

        @pl.when(i > 0)
        def _():
            loss_ref[:, :] = loss_ref[:, :] + part


def _argmin_call(x, e, e2):
    m = x.shape[0]
    grid = (m // _TM, _VOCAB // _TK)
    return pl.pallas_call(
        _argmin_body,
        grid=grid,
        in_specs=[
            pl.BlockSpec((_TM, _DIM), lambda i, k: (i, 0)),
            pl.BlockSpec((_TK, _DIM), lambda i, k: (k, 0)),
            pl.BlockSpec((1, _TK), lambda i, k: (0, k)),
        ],
        out_specs=[
            pl.BlockSpec((_TM, 1), lambda i, k: (i, 0)),
            pl.BlockSpec((1, 1), lambda i, k: (0, 0)),
        ],
        out_shape=[
            jax.ShapeDtypeStruct((m, 1), jnp.int32),
            jax.ShapeDtypeStruct((1, 1), jnp.float32),
        ],
        scratch_shapes=[
            pltpu.VMEM((_TM, 1), jnp.float32),
            pltpu.VMEM((_TM, 1), jnp.float32),
            pltpu.VMEM((1, 128), jnp.float32),
        ],
    )(x, e, e2)


def _perp_body(h_ref, out_ref):
    c = h_ref[0:1, :] + h_ref[1:2, :]
    p = c * (1.0 / 8192.0)
    t = p * jnp.log(p + 1e-10)
    out_ref[:, :] = jnp.exp(-jnp.sum(t, axis=1, keepdims=True))


def _perp_call(hist):
    return pl.pallas_call(
        _perp_body,
        out_shape=jax.ShapeDtypeStruct((1, 1), jnp.float32),
    )(hist)


def _sc_gather_hist(e, idx, zeros_k, ones_h):
    info = plsc.get_sparse_core_info()
    nc, ns = info.num_cores, info.num_subcores
    nw = nc * ns
    m = idx.shape[0]
    bpw = m // nw
    half = bpw // 2
    mesh = plsc.VectorSubcoreMesh(core_axis_name="c", subcore_axis_name="s")

    @functools.partial(
        pl.kernel,
        mesh=mesh,
        out_type=[
            jax.ShapeDtypeStruct((m, _DIM), jnp.float32),
            jax.ShapeDtypeStruct((nc, _VOCAB), jnp.float32),
        ],
        scratch_types=[
            pltpu.VMEM((2, half), jnp.int32),
            pltpu.VMEM((half, _DIM), jnp.float32),
            pltpu.VMEM((half, _DIM), jnp.float32),
            pltpu.VMEM((half,), jnp.float32),
            pltpu.VMEM_SHARED((_VOCAB,), jnp.float32),
            pltpu.SemaphoreType.DMA,
            pltpu.SemaphoreType.DMA,
            pltpu.SemaphoreType.DMA,
            pltpu.SemaphoreType.DMA,
        ],
    )
    def body(e_hbm, idx_hbm, zeros_hbm, ones_hbm, q_hbm, hist_hbm,
             idx_v, rows_a, rows_b, ones_v, hist_s, sem_a, sem_b,
             sem_wa, sem_wb):
        cid = lax.axis_index("c")
        sid = lax.axis_index("s")
        wid = sid * nc + cid
        base = wid * bpw
        pltpu.sync_copy(idx_hbm.at[pl.ds(base, half)], idx_v.at[0])
        pltpu.sync_copy(idx_hbm.at[pl.ds(base + half, half)], idx_v.at[1])
        cp_a = pltpu.async_copy(e_hbm.at[idx_v.at[0]], rows_a, sem_a)
        cp_b = pltpu.async_copy(e_hbm.at[idx_v.at[1]], rows_b, sem_b)
        pltpu.sync_copy(ones_hbm, ones_v)

        @pl.when(sid == 0)
        def _():
            pltpu.sync_copy(zeros_hbm, hist_s)

        cp_a.wait()
        wr_a = pltpu.async_copy(rows_a, q_hbm.at[pl.ds(base, half)], sem_wa)
        cp_b.wait()
        wr_b = pltpu.async_copy(rows_b, q_hbm.at[pl.ds(base + half, half)], sem_wb)
        plsc.subcore_barrier()
        pltpu.sync_copy(ones_v, hist_s.at[idx_v.at[0]], add=True)
        pltpu.sync_copy(ones_v, hist_s.at[idx_v.at[1]], add=True)
        plsc.subcore_barrier()

        @pl.when(sid == 0)
        def _():
            pltpu.sync_copy(hist_s, hist_hbm.at[cid])

        wr_a.wait()
        wr_b.wait()

    return body(e, idx, zeros_k, ones_h)


def kernel(inputs, dictionary, counts):
    c = inputs.shape[-1]
    x = inputs.reshape(-1, c)
    m = x.shape[0]
    e = dictionary / counts[:, None]
    e2 = (e ** 2).sum(axis=1).reshape(1, -1)

    idx, loss_sum = _argmin_call(x, e, e2)

    zeros_k = jnp.zeros((_VOCAB,), jnp.float32)
    ones_h = jnp.ones((m // 64,), jnp.float32)
    quantized, hist = _sc_gather_hist(e, idx.reshape(m), zeros_k, ones_h)

    perp = _perp_call(hist)

    loss_vq = loss_sum.reshape(()) * ((1.0 + _BETA) / (m * c))
    return (quantized.reshape(inputs.shape), loss_vq, perp.reshape(()))
